# Initial kernel scaffold; baseline (speedup 1.0000x reference)
import jax
import jax.numpy as jnp
from jax.experimental import pallas as pl


def kernel(u, v, boundary_norm, is_boundary, is_inflow, y_force, pos, edge_index, W_emb1, b_emb1, W_emb2, b_emb2, ln_g, ln_b, Wm1, bm1, Wm2, bm2, Wu1, bu1, Wu2, bu2, lnf_g, lnf_b, Wr1, br1, Wr2, br2, Ws1, bs1, Ws2, bs2):
    raise NotImplementedError("write your pallas kernel here")


# speedup vs baseline: 1.1810x; 1.1810x over previous
import functools

import jax
import jax.numpy as jnp
from jax import lax
from jax.experimental import pallas as pl
from jax.experimental.pallas import tpu as pltpu
from jax.experimental.pallas import tpu_sc as plsc

N = 100000
E = 1600000
EP = 1638400
IR = EP // 128
HALF = N // 2
DUMP = 12500
ACC_ROWS = 12808
ZROWS = 12800
ZBUF = 8
S = 32
SENTINEL = 2000000


def _ln(x, g, b):
    m = x.mean(-1, keepdims=True)
    var = ((x - m) ** 2).mean(-1, keepdims=True)
    return (x - m) * lax.rsqrt(var + 1e-5) * g + b


def _make_gather(D):
    NW = 32
    rows_w = IR // NW
    CH = 4
    n_chunks = rows_w // CH

    @functools.partial(
        pl.kernel,
        mesh=plsc.VectorSubcoreMesh(core_axis_name="c", subcore_axis_name="s"),
        out_type=jax.ShapeDtypeStruct((EP, D), jnp.float32),
        scratch_types=[
            pltpu.VMEM((CH, 128), jnp.int32),
            pltpu.VMEM((CH * 128, D), jnp.float32),
            pltpu.SemaphoreType.DMA,
        ],
    )
    def gk(table_hbm, idx_hbm, out_hbm, idx_v, rows_v, sem):
        wid = lax.axis_index("s") * 2 + lax.axis_index("c")
        base = wid * rows_w

        def chunk(c, carry):
            pltpu.sync_copy(idx_hbm.at[pl.ds(base + c * CH, CH), :], idx_v)
            for j in range(CH):
                pltpu.async_copy(
                    table_hbm.at[idx_v.at[j]],
                    rows_v.at[pl.ds(j * 128, 128), :],
                    sem,
                ).wait()
            pltpu.sync_copy(
                rows_v, out_hbm.at[pl.ds((base + c * CH) * 128, CH * 128), :]
            )
            return carry

        lax.fori_loop(0, n_chunks, chunk, 0)

    return gk


def _make_scatter():
    rows_c = IR // 16
    CH = 1
    n_chunks = rows_c // CH
    zrows = ZROWS // 16
    nz = zrows // ZBUF

    @functools.partial(
        pl.kernel,
        mesh=plsc.VectorSubcoreMesh(core_axis_name="c", subcore_axis_name="s"),
        out_type=jax.ShapeDtypeStruct((2 * 12504, 128), jnp.float32),
        scratch_types=[
            pltpu.VMEM((CH, 128), jnp.int32),
            pltpu.VMEM((CH * 128, 128), jnp.float32),
            pltpu.VMEM((ZBUF, 128), jnp.float32),
            pltpu.VMEM_SHARED((ACC_ROWS, 128), jnp.float32),
        ],
    )
    def sk(idx_hbm, m_hbm, out_hbm, idx_v, m_v, zero_v, acc_sh):
        cid = lax.axis_index("c")
        sid = lax.axis_index("s")

        z = jnp.zeros((16,), jnp.float32)
        for r in range(ZBUF):
            zero_v[r, pl.ds(0, 16)] = z
            zero_v[r, pl.ds(16, 16)] = z

        def zloop(i, carry):
            pltpu.sync_copy(
                zero_v, acc_sh.at[pl.ds(sid * zrows + i * ZBUF, ZBUF), :]
            )
            return carry

        lax.fori_loop(0, nz, zloop, 0)
        plsc.subcore_barrier()

        base = sid * rows_c

        def chunk(c, carry):
            pltpu.sync_copy(idx_hbm.at[cid, pl.ds(base + c * CH, CH), :], idx_v)
            pltpu.sync_copy(m_hbm.at[pl.ds((base + c * CH) * 128, CH * 128), :], m_v)
            for j in range(CH):
                pltpu.sync_copy(
                    m_v.at[pl.ds(j * 128, 128), :],
                    acc_sh.at[idx_v.at[j]],
                    add=True,
                )
            return carry

        lax.fori_loop(0, n_chunks, chunk, 0)
        plsc.subcore_barrier()

        half_q = 12504
        ra = 784
        rb = 744

        @pl.when(sid < 15)
        def _():
            pltpu.sync_copy(
                acc_sh.at[pl.ds(sid * ra, ra), :],
                out_hbm.at[pl.ds(cid * half_q + sid * ra, ra), :],
            )

        @pl.when(sid == 15)
        def _():
            pltpu.sync_copy(
                acc_sh.at[pl.ds(15 * ra, rb), :],
                out_hbm.at[pl.ds(cid * half_q + 15 * ra, rb), :],
            )

    return sk


_BN = 2000
_BE = 4096


def _node_spec(width):
    return pl.BlockSpec((_BN, width), lambda i: (i, 0))


def _edge_spec(width):
    return pl.BlockSpec((_BE, width), lambda i: (i, 0))


def _full_spec(shape):
    return pl.BlockSpec(shape, lambda i: tuple(0 for _ in shape))


def _embed_body(u_r, vx_r, vy_r, rest_r, pos_r, wa, wb, wc, b1, w2, b2, g, bb,
                x_o, h_o, t_o):
    vn = jnp.sqrt(vx_r[...] ** 2 + vy_r[...] ** 2)
    pre = u_r[...] @ wa[...] + vn @ wb[...] + rest_r[...] @ wc[...] + b1[...]
    x = jax.nn.silu(pre) @ w2[...] + b2[...]
    x_o[...] = x
    h = _ln(x, g[...], bb[...])
    h_o[...] = h
    t_o[...] = jnp.concatenate(
        [h, pos_r[...], jnp.zeros((h.shape[0], 94), jnp.float32)], axis=1
    )


def _previdx_body(c_r, i0_o, i1_o):
    c = c_r[...]
    i0_o[...] = jnp.where(c < HALF, c // 4, DUMP)
    i1_o[...] = jnp.where((c >= HALF) & (c < N), (c - HALF) // 4, DUMP)


def _place4(m, cm):
    return jnp.concatenate(
        [m * (cm == k).astype(jnp.float32) for k in range(4)], axis=1
    )


def _edge0_body(hr48_r, hc48_r, cm_r, w1a, w1b, w1c, b1, w2, b2, m_o, de_o):
    hr48 = hr48_r[...]
    hc48 = hc48_r[...]
    hr = hr48[:, :32]
    hc = hc48[:, :32]
    dx = hr48[:, 32:33] - hc48[:, 32:33]
    dy = hr48[:, 33:34] - hc48[:, 33:34]
    d = jnp.sqrt(dx * dx + dy * dy)
    dc = jnp.maximum(d, 1e-6)
    theta = jnp.pi * dc
    s1 = jnp.sin(theta)
    c2 = 2.0 * jnp.cos(theta)
    sins = [s1, c2 * s1]
    for _ in range(14):
        sins.append(c2 * sins[-1] - sins[-2])
    de = jnp.sqrt(2.0) * jnp.concatenate(sins, axis=1) / dc
    de_o[...] = de
    pre = hr @ w1a[...] + hc @ w1b[...] + de @ w1c[...] + b1[...]
    m = jax.nn.silu(pre) @ w2[...] + b2[...]
    m_o[...] = _place4(m, cm_r[...])


def _edge_body(hr_r, hc_r, de_r, cm_r, w1a, w1b, w1c, b1, w2, b2, m_o):
    pre = (hr_r[...][:, :32] @ w1a[...] + hc_r[...][:, :32] @ w1b[...]
           + de_r[...] @ w1c[...] + b1[...])
    m = jax.nn.silu(pre) @ w2[...] + b2[...]
    m_o[...] = _place4(m, cm_r[...])


def _upd_body(x_r, h_r, agg_r, w1a, w1b, b1, w2, b2, g, bb, x_o, h_o):
    pre = h_r[...][:, :32] @ w1a[...] + agg_r[...] @ w1b[...] + b1[...]
    xn = x_r[...] + jax.nn.silu(pre) @ w2[...] + b2[...]
    x_o[...] = xn
    hn = _ln(xn, g[...], bb[...])
    h_o[...] = jnp.concatenate(
        [hn, jnp.zeros((hn.shape[0], 96), jnp.float32)], axis=1
    )


def _final_body(h_r, wr1, br1, wr2, br2, ws1, bs1, ws2, bs2, ul, vl, u_o, v_o):
    h = h_r[...][:, :32]
    v_o[...] = jax.nn.silu(h @ wr1[...] + br1[...]) @ wr2[...] + br2[...] + vl[...]
    u_o[...] = jax.nn.silu(h @ ws1[...] + bs1[...]) @ ws2[...] + bs2[...] + ul[...]


def kernel(u, v, boundary_norm, is_boundary, is_inflow, y_force, pos,
           edge_index, W_emb1, b_emb1, W_emb2, b_emb2, ln_g, ln_b, Wm1, bm1,
           Wm2, bm2, Wu1, bu1, Wu2, bu2, lnf_g, lnf_b, Wr1, br1, Wr2, br2,
           Ws1, bs1, Ws2, bs2):
    f32 = jnp.float32
    row = edge_index[0]
    col = edge_index[1]
    padz = jnp.zeros((EP - E,), jnp.int32)
    rowp = jnp.concatenate([row, padz]).reshape(IR, 128)
    colg = jnp.concatenate([col, padz]).reshape(IR, 128)
    cols = jnp.concatenate(
        [col, jnp.full((EP - E,), SENTINEL, jnp.int32)]
    ).reshape(IR, 128)

    grid_n = N // _BN
    grid_e = EP // _BE

    cm4 = jnp.concatenate(
        [(col % 4).astype(f32), jnp.zeros((EP - E,), f32)]
    ).reshape(EP, 1)

    i0, i1 = pl.pallas_call(
        _previdx_body,
        grid=(IR // 1280,),
        in_specs=[pl.BlockSpec((1280, 128), lambda i: (i, 0))],
        out_specs=[pl.BlockSpec((1280, 128), lambda i: (i, 0))] * 2,
        out_shape=[jax.ShapeDtypeStruct((IR, 128), jnp.int32)] * 2,
    )(cols)
    idx2 = jnp.stack([i0, i1])

    vx = v[:, :, 0]
    vy = v[:, :, 1]
    rest = jnp.concatenate([boundary_norm, is_inflow, y_force], axis=1)
    x0, h0, t48 = pl.pallas_call(
        _embed_body,
        grid=(grid_n,),
        in_specs=[
            _node_spec(8), _node_spec(8), _node_spec(8), _node_spec(4),
            _node_spec(2),
            _full_spec((8, S)), _full_spec((8, S)), _full_spec((4, S)),
            _full_spec((1, S)), _full_spec((S, S)), _full_spec((1, S)),
            _full_spec((1, S)), _full_spec((1, S)),
        ],
        out_specs=[_node_spec(S), _node_spec(S), _node_spec(128)],
        out_shape=[
            jax.ShapeDtypeStruct((N, S), f32),
            jax.ShapeDtypeStruct((N, S), f32),
            jax.ShapeDtypeStruct((N, 128), f32),
        ],
    )(u, vx, vy, rest, pos,
      W_emb1[:8], W_emb1[8:16], W_emb1[16:20], b_emb1.reshape(1, S),
      W_emb2, b_emb2.reshape(1, S),
      ln_g[0].reshape(1, S), ln_b[0].reshape(1, S))

    g128 = _make_gather(128)
    scat = _make_scatter()

    x, h = x0, h0
    de = None
    L = Wm1.shape[0]
    for i in range(L):
        if i == 0:
            hr48 = g128(t48, rowp)
            hc48 = g128(t48, colg)
            m, de = pl.pallas_call(
                _edge0_body,
                grid=(grid_e,),
                in_specs=[
                    _edge_spec(128), _edge_spec(128), _edge_spec(1),
                    _full_spec((S, S)), _full_spec((S, S)),
                    _full_spec((16, S)), _full_spec((1, S)),
                    _full_spec((S, S)), _full_spec((1, S)),
                ],
                out_specs=[_edge_spec(128), _edge_spec(16)],
                out_shape=[
                    jax.ShapeDtypeStruct((EP, 128), f32),
                    jax.ShapeDtypeStruct((EP, 16), f32),
                ],
            )(hr48, hc48, cm4,
              Wm1[i, :S], Wm1[i, S:2 * S], Wm1[i, 2 * S:],
              bm1[i].reshape(1, S), Wm2[i], bm2[i].reshape(1, S))
        else:
            hr = g128(h, rowp)
            hc = g128(h, colg)
            m = pl.pallas_call(
                _edge_body,
                grid=(grid_e,),
                in_specs=[
                    _edge_spec(128), _edge_spec(128), _edge_spec(16),
                    _edge_spec(1),
                    _full_spec((S, S)), _full_spec((S, S)),
                    _full_spec((16, S)), _full_spec((1, S)),
                    _full_spec((S, S)), _full_spec((1, S)),
                ],
                out_specs=_edge_spec(128),
                out_shape=jax.ShapeDtypeStruct((EP, 128), f32),
            )(hr, hc, de, cm4,
              Wm1[i, :S], Wm1[i, S:2 * S], Wm1[i, 2 * S:],
              bm1[i].reshape(1, S), Wm2[i], bm2[i].reshape(1, S))

        o = scat(idx2, m)
        agg = jnp.concatenate([o[:12500], o[12504:25004]], axis=0).reshape(N, S)

        gn = ln_g[i + 1].reshape(1, S) if i + 1 < L else lnf_g.reshape(1, S)
        bn = ln_b[i + 1].reshape(1, S) if i + 1 < L else lnf_b.reshape(1, S)
        x, h = pl.pallas_call(
            _upd_body,
            grid=(grid_n,),
            in_specs=[
                _node_spec(S), _node_spec(128), _node_spec(S),
                _full_spec((S, S)), _full_spec((S, S)), _full_spec((1, S)),
                _full_spec((S, S)), _full_spec((1, S)),
                _full_spec((1, S)), _full_spec((1, S)),
            ],
            out_specs=[_node_spec(S), _node_spec(128)],
            out_shape=[
                jax.ShapeDtypeStruct((N, S), f32),
                jax.ShapeDtypeStruct((N, 128), f32),
            ],
        )(x, h, agg,
          Wu1[i, :S], Wu1[i, S:], bu1[i].reshape(1, S),
          Wu2[i], bu2[i].reshape(1, S), gn, bn)

    uo, vo = pl.pallas_call(
        _final_body,
        grid=(grid_n,),
        in_specs=[
            _node_spec(128),
            _full_spec((S, 3 * S)), _full_spec((1, 3 * S)),
            _full_spec((3 * S, 2)), _full_spec((1, 2)),
            _full_spec((S, 3 * S)), _full_spec((1, 3 * S)),
            _full_spec((3 * S, 1)), _full_spec((1, 1)),
            _node_spec(1), _node_spec(2),
        ],
        out_specs=[_node_spec(1), _node_spec(2)],
        out_shape=[
            jax.ShapeDtypeStruct((N, 1), f32),
            jax.ShapeDtypeStruct((N, 2), f32),
        ],
    )(h, Wr1, br1.reshape(1, 3 * S), Wr2, br2.reshape(1, 2),
      Ws1, bs1.reshape(1, 3 * S), Ws2, bs2.reshape(1, 1),
      u[:, -1:], v[:, -1, :])

    return (uo.reshape(-1), vo)
